# baseline (speedup 1.0000x reference)
import functools

import jax
import jax.numpy as jnp
from jax import lax
from jax.experimental import pallas as pl
from jax.experimental.pallas import tpu as pltpu
from jax.experimental.pallas import tpu_sc as plsc

NC = 2
NS = 16
NW = NC * NS

N = 10000
E = 320000
D = 128
H = 128
C = 16

NP = 10240
RPT = NP // NS
EP = NW * 80 * 128
KB = EP // NW // 128
ZR = 64
NBUF = 4


def _mesh():
    return plsc.VectorSubcoreMesh(core_axis_name="c", subcore_axis_name="s")


DW = 16


@functools.partial(
    pl.kernel,
    out_type=jax.ShapeDtypeStruct((NC, NP, DW), jnp.float32),
    mesh=_mesh(),
    compiler_params=pltpu.CompilerParams(use_tc_tiling_on_sc=False),
    scratch_types=[
        pltpu.VMEM((KB, 128), jnp.int32),
        pltpu.VMEM((128, DW), jnp.float32),
        pltpu.VMEM((ZR, DW), jnp.float32),
        pltpu.VMEM_SHARED((NP, DW), jnp.float32),
    ],
)
def _sc_degree(dst_hbm, out_hbm, dst_v, ones_v, zbuf, acc):
    c = lax.axis_index("c")
    s = lax.axis_index("s")
    w = c * NS + s
    pltpu.sync_copy(dst_hbm.at[pl.ds(w * KB, KB)], dst_v)

    zero16 = jnp.zeros((16,), jnp.float32)
    one16 = jnp.ones((16,), jnp.float32)

    @pl.loop(0, 128)
    def _(r):
        ones_v[r, :] = one16

    @pl.loop(0, ZR)
    def _(r):
        zbuf[r, :] = zero16

    @pl.loop(0, RPT // ZR)
    def _(i):
        pltpu.sync_copy(zbuf, acc.at[pl.ds(s * RPT + i * ZR, ZR)])

    plsc.subcore_barrier()

    @pl.loop(0, KB)
    def _(j):
        pltpu.sync_copy(ones_v, acc.at[dst_v.at[j]], add=True)

    plsc.subcore_barrier()

    @pl.loop(0, RPT // ZR)
    def _(i):
        sl = pl.ds(s * RPT + i * ZR, ZR)
        pltpu.sync_copy(acc.at[sl], out_hbm.at[c, sl])


FH = H // NC
KB2 = EP // NS // 128


@functools.partial(
    pl.kernel,
    out_type=jax.ShapeDtypeStruct((NC, NP, FH), jnp.float32),
    mesh=_mesh(),
    compiler_params=pltpu.CompilerParams(use_tc_tiling_on_sc=False),
    scratch_types=[
        pltpu.VMEM((KB2, 128), jnp.int32),
        pltpu.VMEM((KB2, 128), jnp.int32),
        pltpu.VMEM((NBUF, 128, FH), jnp.float32),
        pltpu.VMEM((ZR, FH), jnp.float32),
        pltpu.VMEM_SHARED((NP, FH), jnp.float32),
        [pltpu.SemaphoreType.DMA] * NBUF,
        [pltpu.SemaphoreType.DMA] * NBUF,
    ],
)
def _sc_agg128(z_hbm, src_hbm, dst_hbm, out_hbm,
               src_v, dst_v, rows, zbuf, acc, gsems, ssems):
    c = lax.axis_index("c")
    s = lax.axis_index("s")
    zc = z_hbm.at[c]
    pltpu.sync_copy(src_hbm.at[pl.ds(s * KB2, KB2)], src_v)
    pltpu.sync_copy(dst_hbm.at[pl.ds(s * KB2, KB2)], dst_v)

    zero16 = jnp.zeros((16,), jnp.float32)

    @pl.loop(0, ZR)
    def _(r):
        @pl.loop(0, FH // 16)
        def _(k):
            zbuf[r, pl.ds(k * 16, 16)] = zero16

    @pl.loop(0, RPT // ZR)
    def _(i):
        pltpu.sync_copy(zbuf, acc.at[pl.ds(s * RPT + i * ZR, ZR)])

    plsc.subcore_barrier()

    for b in range(NBUF):
        pltpu.async_copy(zc.at[src_v.at[b]], rows.at[b], gsems[b])

    @pl.loop(0, KB2 // NBUF)
    def _(g):
        j0 = NBUF * g
        for b in range(NBUF):
            pltpu.make_async_copy(
                zc.at[src_v.at[j0 + b]], rows.at[b], gsems[b]).wait()
            pltpu.async_copy(rows.at[b], acc.at[dst_v.at[j0 + b]],
                             ssems[b], add=True)
        for b in range(NBUF):
            @pl.when(j0 + b + NBUF < KB2)
            def _():
                pltpu.make_async_copy(
                    rows.at[b], acc.at[dst_v.at[j0 + b]], ssems[b]).wait()
                pltpu.async_copy(
                    zc.at[src_v.at[j0 + b + NBUF]], rows.at[b], gsems[b])

    for b in range(NBUF):
        pltpu.make_async_copy(
            rows.at[b], acc.at[dst_v.at[KB2 - NBUF + b]], ssems[b]).wait()

    plsc.subcore_barrier()

    @pl.loop(0, RPT // ZR)
    def _(i):
        sl = pl.ds(s * RPT + i * ZR, ZR)
        pltpu.sync_copy(acc.at[sl], out_hbm.at[c, sl])


def _make_sc_agg(F):
    @functools.partial(
        pl.kernel,
        out_type=jax.ShapeDtypeStruct((NC, NP, F), jnp.float32),
        mesh=_mesh(),
        compiler_params=pltpu.CompilerParams(use_tc_tiling_on_sc=False),
        scratch_types=[
            pltpu.VMEM((KB, 128), jnp.int32),
            pltpu.VMEM((KB, 128), jnp.int32),
            pltpu.VMEM((NBUF, 128, F), jnp.float32),
            pltpu.VMEM((ZR, F), jnp.float32),
            pltpu.VMEM_SHARED((NP, F), jnp.float32),
            [pltpu.SemaphoreType.DMA] * NBUF,
            [pltpu.SemaphoreType.DMA] * NBUF,
        ],
    )
    def agg(z_hbm, src_hbm, dst_hbm, out_hbm,
            src_v, dst_v, rows, zbuf, acc, gsems, ssems):
        c = lax.axis_index("c")
        s = lax.axis_index("s")
        w = c * NS + s
        pltpu.sync_copy(src_hbm.at[pl.ds(w * KB, KB)], src_v)
        pltpu.sync_copy(dst_hbm.at[pl.ds(w * KB, KB)], dst_v)

        zero16 = jnp.zeros((16,), jnp.float32)

        @pl.loop(0, ZR)
        def _(r):
            @pl.loop(0, F // 16)
            def _(k):
                zbuf[r, pl.ds(k * 16, 16)] = zero16

        @pl.loop(0, RPT // ZR)
        def _(i):
            pltpu.sync_copy(zbuf, acc.at[pl.ds(s * RPT + i * ZR, ZR)])

        plsc.subcore_barrier()

        for b in range(NBUF):
            pltpu.async_copy(z_hbm.at[src_v.at[b]], rows.at[b], gsems[b])

        @pl.loop(0, KB // NBUF)
        def _(g):
            j0 = NBUF * g
            for b in range(NBUF):
                pltpu.make_async_copy(
                    z_hbm.at[src_v.at[j0 + b]], rows.at[b], gsems[b]).wait()
                pltpu.async_copy(rows.at[b], acc.at[dst_v.at[j0 + b]],
                                 ssems[b], add=True)
            for b in range(NBUF):
                @pl.when(j0 + b + NBUF < KB)
                def _():
                    pltpu.make_async_copy(
                        rows.at[b], acc.at[dst_v.at[j0 + b]], ssems[b]).wait()
                    pltpu.async_copy(
                        z_hbm.at[src_v.at[j0 + b + NBUF]], rows.at[b], gsems[b])

        for b in range(NBUF):
            pltpu.make_async_copy(
                rows.at[b], acc.at[dst_v.at[KB - NBUF + b]], ssems[b]).wait()

        plsc.subcore_barrier()

        @pl.loop(0, RPT // ZR)
        def _(i):
            sl = pl.ds(s * RPT + i * ZR, ZR)
            pltpu.sync_copy(acc.at[sl], out_hbm.at[c, sl])

    return agg


_sc_agg16 = _make_sc_agg(C)


BLK = 1024


def _tc_pre_body(d0_ref, d1_ref, x_ref, w_ref, isd_ref, z_ref):
    deg = d0_ref[...] + d1_ref[...] + 1.0
    isd = lax.rsqrt(deg)
    isd_ref[...] = isd
    z = jnp.dot(x_ref[...] * isd[:, None], w_ref[...],
                preferred_element_type=jnp.float32)
    z_ref[0] = z[:, :FH]
    z_ref[1] = z[:, FH:]


def _tc_pre(deg0, deg1, x_pad, W1):
    return pl.pallas_call(
        _tc_pre_body,
        grid=(NP // BLK,),
        in_specs=[
            pl.BlockSpec((BLK,), lambda i: (i,)),
            pl.BlockSpec((BLK,), lambda i: (i,)),
            pl.BlockSpec((BLK, D), lambda i: (i, 0)),
            pl.BlockSpec((D, H), lambda i: (0, 0)),
        ],
        out_specs=[
            pl.BlockSpec((BLK,), lambda i: (i,)),
            pl.BlockSpec((NC, BLK, FH), lambda i: (0, i, 0)),
        ],
        out_shape=[
            jax.ShapeDtypeStruct((NP,), jnp.float32),
            jax.ShapeDtypeStruct((NC, NP, FH), jnp.float32),
        ],
    )(deg0, deg1, x_pad, W1)


def _tc_mid_body(isd_ref, p_ref, b1_ref, w2_ref, z2_ref):
    isd = isd_ref[...]
    agg = jnp.concatenate([p_ref[0], p_ref[1]], axis=1)
    h = jnp.maximum(agg * isd[:, None] + b1_ref[...][None, :], 0.0)
    z2_ref[...] = jnp.dot(h * isd[:, None], w2_ref[...],
                          preferred_element_type=jnp.float32)


def _tc_mid(isd, p, b1, W2):
    return pl.pallas_call(
        _tc_mid_body,
        grid=(NP // BLK,),
        in_specs=[
            pl.BlockSpec((BLK,), lambda i: (i,)),
            pl.BlockSpec((NC, BLK, FH), lambda i: (0, i, 0)),
            pl.BlockSpec((H,), lambda i: (0,)),
            pl.BlockSpec((H, C), lambda i: (0, 0)),
        ],
        out_specs=pl.BlockSpec((BLK, C), lambda i: (i, 0)),
        out_shape=jax.ShapeDtypeStruct((NP, C), jnp.float32),
    )(isd, p, b1, W2)


def _tc_post_body(isd_ref, p0_ref, p1_ref, b2_ref, out_ref):
    logits = ((p0_ref[...] + p1_ref[...]) * isd_ref[...][:, None]
              + b2_ref[...][None, :])
    m = jnp.max(logits, axis=-1, keepdims=True)
    e = jnp.exp(logits - m)
    out_ref[...] = e / jnp.sum(e, axis=-1, keepdims=True)


def _tc_post(isd, p0, p1, b2):
    return pl.pallas_call(
        _tc_post_body,
        grid=(NP // BLK,),
        in_specs=[
            pl.BlockSpec((BLK,), lambda i: (i,)),
            pl.BlockSpec((BLK, C), lambda i: (i, 0)),
            pl.BlockSpec((BLK, C), lambda i: (i, 0)),
            pl.BlockSpec((C,), lambda i: (0,)),
        ],
        out_specs=pl.BlockSpec((BLK, C), lambda i: (i, 0)),
        out_shape=jax.ShapeDtypeStruct((NP, C), jnp.float32),
    )(isd, p0, p1, b2)


@jax.jit
def kernel(node_embeddings, adjacency_lists, W1, b1, W2, b2):
    src = adjacency_lists[0]
    dst = adjacency_lists[1]
    pad = jnp.full((EP - E,), N, jnp.int32)
    src2d = jnp.concatenate([src, pad]).reshape(EP // 128, 128)
    dst2d = jnp.concatenate([dst, pad]).reshape(EP // 128, 128)
    x_pad = jnp.pad(node_embeddings, ((0, NP - N), (0, 0)))

    degp = _sc_degree(dst2d)[:, :, 0]
    isd, z1 = _tc_pre(degp[0], degp[1], x_pad, W1)
    agg1 = _sc_agg128(z1, src2d, dst2d)
    z2 = _tc_mid(isd, agg1, b1, W2)
    agg2 = _sc_agg16(z2, src2d, dst2d)
    out = _tc_post(isd, agg2[0], agg2[1], b2)
    return out[:N]


# speedup vs baseline: 1.1060x; 1.1060x over previous
import functools

import jax
import jax.numpy as jnp
from jax import lax
from jax.experimental import pallas as pl
from jax.experimental.pallas import tpu as pltpu
from jax.experimental.pallas import tpu_sc as plsc

NC = 2
NS = 16
NW = NC * NS

N = 10000
E = 320000
D = 128
H = 128
C = 16

NP = 10240
RPT = NP // NS
EP = NW * 80 * 128
KB = EP // NW // 128
ZR = 64
NBUF = 4


def _mesh():
    return plsc.VectorSubcoreMesh(core_axis_name="c", subcore_axis_name="s")


DW = 16


@functools.partial(
    pl.kernel,
    out_type=jax.ShapeDtypeStruct((NC, NP, DW), jnp.float32),
    mesh=_mesh(),
    compiler_params=pltpu.CompilerParams(use_tc_tiling_on_sc=False),
    scratch_types=[
        pltpu.VMEM((KB, 128), jnp.int32),
        pltpu.VMEM((128, DW), jnp.float32),
        pltpu.VMEM((ZR, DW), jnp.float32),
        pltpu.VMEM_SHARED((NP, DW), jnp.float32),
    ],
)
def _sc_degree(dst_hbm, out_hbm, dst_v, ones_v, zbuf, acc):
    c = lax.axis_index("c")
    s = lax.axis_index("s")
    w = c * NS + s
    pltpu.sync_copy(dst_hbm.at[pl.ds(w * KB, KB)], dst_v)

    zero16 = jnp.zeros((16,), jnp.float32)
    one16 = jnp.ones((16,), jnp.float32)

    @pl.loop(0, 128)
    def _(r):
        ones_v[r, :] = one16

    @pl.loop(0, ZR)
    def _(r):
        zbuf[r, :] = zero16

    @pl.loop(0, RPT // ZR)
    def _(i):
        pltpu.sync_copy(zbuf, acc.at[pl.ds(s * RPT + i * ZR, ZR)])

    plsc.subcore_barrier()

    @pl.loop(0, KB)
    def _(j):
        pltpu.sync_copy(ones_v, acc.at[dst_v.at[j]], add=True)

    plsc.subcore_barrier()

    @pl.loop(0, RPT // ZR)
    def _(i):
        sl = pl.ds(s * RPT + i * ZR, ZR)
        pltpu.sync_copy(acc.at[sl], out_hbm.at[c, sl])


FH = H // NC
KB2 = EP // NS // 128


@functools.partial(
    pl.kernel,
    out_type=jax.ShapeDtypeStruct((NC, NP, H), jnp.bfloat16),
    mesh=_mesh(),
    compiler_params=pltpu.CompilerParams(use_tc_tiling_on_sc=False),
    scratch_types=[
        pltpu.VMEM((KB, 128), jnp.int32),
        pltpu.VMEM((KB, 128), jnp.int32),
        pltpu.VMEM((NBUF, 128, H), jnp.bfloat16),
        pltpu.VMEM((ZR, H), jnp.bfloat16),
        pltpu.VMEM_SHARED((NP, H), jnp.bfloat16),
        [pltpu.SemaphoreType.DMA] * NBUF,
        [pltpu.SemaphoreType.DMA] * NBUF,
    ],
)
def _sc_agg128(z_hbm, src_hbm, dst_hbm, out_hbm,
               src_v, dst_v, rows, zbuf, acc, gsems, ssems):
    c = lax.axis_index("c")
    s = lax.axis_index("s")
    w = c * NS + s
    pltpu.sync_copy(src_hbm.at[pl.ds(w * KB, KB)], src_v)
    pltpu.sync_copy(dst_hbm.at[pl.ds(w * KB, KB)], dst_v)

    zero32 = jnp.zeros((32,), jnp.bfloat16)

    @pl.loop(0, ZR)
    def _(r):
        @pl.loop(0, H // 32)
        def _(k):
            zbuf[r, pl.ds(k * 32, 32)] = zero32

    @pl.loop(0, RPT // ZR)
    def _(i):
        pltpu.sync_copy(zbuf, acc.at[pl.ds(s * RPT + i * ZR, ZR)])

    plsc.subcore_barrier()

    for b in range(NBUF):
        pltpu.async_copy(z_hbm.at[src_v.at[b]], rows.at[b], gsems[b])

    @pl.loop(0, KB // NBUF)
    def _(g):
        j0 = NBUF * g
        for b in range(NBUF):
            pltpu.make_async_copy(
                z_hbm.at[src_v.at[j0 + b]], rows.at[b], gsems[b]).wait()
            pltpu.async_copy(rows.at[b], acc.at[dst_v.at[j0 + b]],
                             ssems[b], add=True)
        for b in range(NBUF):
            @pl.when(j0 + b + NBUF < KB)
            def _():
                pltpu.make_async_copy(
                    rows.at[b], acc.at[dst_v.at[j0 + b]], ssems[b]).wait()
                pltpu.async_copy(
                    z_hbm.at[src_v.at[j0 + b + NBUF]], rows.at[b], gsems[b])

    for b in range(NBUF):
        pltpu.make_async_copy(
            rows.at[b], acc.at[dst_v.at[KB - NBUF + b]], ssems[b]).wait()

    plsc.subcore_barrier()

    @pl.loop(0, RPT // ZR)
    def _(i):
        sl = pl.ds(s * RPT + i * ZR, ZR)
        pltpu.sync_copy(acc.at[sl], out_hbm.at[c, sl])


def _make_sc_agg(F):
    @functools.partial(
        pl.kernel,
        out_type=jax.ShapeDtypeStruct((NC, NP, F), jnp.float32),
        mesh=_mesh(),
        compiler_params=pltpu.CompilerParams(use_tc_tiling_on_sc=False),
        scratch_types=[
            pltpu.VMEM((KB, 128), jnp.int32),
            pltpu.VMEM((KB, 128), jnp.int32),
            pltpu.VMEM((NBUF, 128, F), jnp.float32),
            pltpu.VMEM((ZR, F), jnp.float32),
            pltpu.VMEM_SHARED((NP, F), jnp.float32),
            [pltpu.SemaphoreType.DMA] * NBUF,
            [pltpu.SemaphoreType.DMA] * NBUF,
        ],
    )
    def agg(z_hbm, src_hbm, dst_hbm, out_hbm,
            src_v, dst_v, rows, zbuf, acc, gsems, ssems):
        c = lax.axis_index("c")
        s = lax.axis_index("s")
        w = c * NS + s
        pltpu.sync_copy(src_hbm.at[pl.ds(w * KB, KB)], src_v)
        pltpu.sync_copy(dst_hbm.at[pl.ds(w * KB, KB)], dst_v)

        zero16 = jnp.zeros((16,), jnp.float32)

        @pl.loop(0, ZR)
        def _(r):
            @pl.loop(0, F // 16)
            def _(k):
                zbuf[r, pl.ds(k * 16, 16)] = zero16

        @pl.loop(0, RPT // ZR)
        def _(i):
            pltpu.sync_copy(zbuf, acc.at[pl.ds(s * RPT + i * ZR, ZR)])

        plsc.subcore_barrier()

        for b in range(NBUF):
            pltpu.async_copy(z_hbm.at[src_v.at[b]], rows.at[b], gsems[b])

        @pl.loop(0, KB // NBUF)
        def _(g):
            j0 = NBUF * g
            for b in range(NBUF):
                pltpu.make_async_copy(
                    z_hbm.at[src_v.at[j0 + b]], rows.at[b], gsems[b]).wait()
                pltpu.async_copy(rows.at[b], acc.at[dst_v.at[j0 + b]],
                                 ssems[b], add=True)
            for b in range(NBUF):
                @pl.when(j0 + b + NBUF < KB)
                def _():
                    pltpu.make_async_copy(
                        rows.at[b], acc.at[dst_v.at[j0 + b]], ssems[b]).wait()
                    pltpu.async_copy(
                        z_hbm.at[src_v.at[j0 + b + NBUF]], rows.at[b], gsems[b])

        for b in range(NBUF):
            pltpu.make_async_copy(
                rows.at[b], acc.at[dst_v.at[KB - NBUF + b]], ssems[b]).wait()

        plsc.subcore_barrier()

        @pl.loop(0, RPT // ZR)
        def _(i):
            sl = pl.ds(s * RPT + i * ZR, ZR)
            pltpu.sync_copy(acc.at[sl], out_hbm.at[c, sl])

    return agg


_sc_agg16 = _make_sc_agg(C)


BLK = 1024


def _tc_pre_body(d0_ref, d1_ref, x_ref, w_ref, isd_ref, z_ref):
    deg = d0_ref[...] + d1_ref[...] + 1.0
    isd = lax.rsqrt(deg)
    isd_ref[...] = isd
    z = jnp.dot(x_ref[...] * isd[:, None], w_ref[...],
                preferred_element_type=jnp.float32)
    z_ref[...] = z.astype(jnp.bfloat16)


def _tc_pre(deg0, deg1, x_pad, W1):
    return pl.pallas_call(
        _tc_pre_body,
        grid=(NP // BLK,),
        in_specs=[
            pl.BlockSpec((BLK,), lambda i: (i,)),
            pl.BlockSpec((BLK,), lambda i: (i,)),
            pl.BlockSpec((BLK, D), lambda i: (i, 0)),
            pl.BlockSpec((D, H), lambda i: (0, 0)),
        ],
        out_specs=[
            pl.BlockSpec((BLK,), lambda i: (i,)),
            pl.BlockSpec((BLK, H), lambda i: (i, 0)),
        ],
        out_shape=[
            jax.ShapeDtypeStruct((NP,), jnp.float32),
            jax.ShapeDtypeStruct((NP, H), jnp.bfloat16),
        ],
    )(deg0, deg1, x_pad, W1)


def _tc_mid_body(isd_ref, p_ref, b1_ref, w2_ref, z2_ref):
    isd = isd_ref[...]
    agg = (p_ref[0].astype(jnp.float32) + p_ref[1].astype(jnp.float32))
    h = jnp.maximum(agg * isd[:, None] + b1_ref[...][None, :], 0.0)
    z2_ref[...] = jnp.dot(h * isd[:, None], w2_ref[...],
                          preferred_element_type=jnp.float32)


def _tc_mid(isd, p, b1, W2):
    return pl.pallas_call(
        _tc_mid_body,
        grid=(NP // BLK,),
        in_specs=[
            pl.BlockSpec((BLK,), lambda i: (i,)),
            pl.BlockSpec((NC, BLK, H), lambda i: (0, i, 0)),
            pl.BlockSpec((H,), lambda i: (0,)),
            pl.BlockSpec((H, C), lambda i: (0, 0)),
        ],
        out_specs=pl.BlockSpec((BLK, C), lambda i: (i, 0)),
        out_shape=jax.ShapeDtypeStruct((NP, C), jnp.float32),
    )(isd, p, b1, W2)


def _tc_post_body(isd_ref, p0_ref, p1_ref, b2_ref, out_ref):
    logits = ((p0_ref[...] + p1_ref[...]) * isd_ref[...][:, None]
              + b2_ref[...][None, :])
    m = jnp.max(logits, axis=-1, keepdims=True)
    e = jnp.exp(logits - m)
    out_ref[...] = e / jnp.sum(e, axis=-1, keepdims=True)


def _tc_post(isd, p0, p1, b2):
    return pl.pallas_call(
        _tc_post_body,
        grid=(NP // BLK,),
        in_specs=[
            pl.BlockSpec((BLK,), lambda i: (i,)),
            pl.BlockSpec((BLK, C), lambda i: (i, 0)),
            pl.BlockSpec((BLK, C), lambda i: (i, 0)),
            pl.BlockSpec((C,), lambda i: (0,)),
        ],
        out_specs=pl.BlockSpec((BLK, C), lambda i: (i, 0)),
        out_shape=jax.ShapeDtypeStruct((NP, C), jnp.float32),
    )(isd, p0, p1, b2)


@jax.jit
def kernel(node_embeddings, adjacency_lists, W1, b1, W2, b2):
    src = adjacency_lists[0]
    dst = adjacency_lists[1]
    pad = jnp.full((EP - E,), N, jnp.int32)
    src2d = jnp.concatenate([src, pad]).reshape(EP // 128, 128)
    dst2d = jnp.concatenate([dst, pad]).reshape(EP // 128, 128)
    x_pad = jnp.pad(node_embeddings, ((0, NP - N), (0, 0)))

    degp = _sc_degree(dst2d)[:, :, 0]
    isd, z1 = _tc_pre(degp[0], degp[1], x_pad, W1)
    agg1 = _sc_agg128(z1, src2d, dst2d)
    z2 = _tc_mid(isd, agg1, b1, W2)
    agg2 = _sc_agg16(z2, src2d, dst2d)
    out = _tc_post(isd, agg2[0], agg2[1], b2)
    return out[:N]
